# baseline (speedup 1.0000x reference)
import functools

import jax
import jax.numpy as jnp
from jax import lax
from jax.experimental import pallas as pl
from jax.experimental.pallas import tpu as pltpu
from jax.experimental.pallas import tpu_sc as plsc

N = 50000
D = 64
E = 800000

NC = 2
NS = 16
NH = 26624
SLAB = NH // NS
ET = 51200
EPAD = NS * ET
B = 128
NBATCH = ET // B


def _sc_body(x_h, src_h, dst_h, z2_h, z1_h, one_h, agg_h, cnt_h,
             srcb, dstb, gidx, sidx, rows, onev, acc, cnta, sem):
    c = lax.axis_index("c")
    s = lax.axis_index("s")
    lo = c * NH
    slab = s * SLAB

    pltpu.sync_copy(z2_h, acc.at[pl.ds(slab, SLAB)])
    pltpu.sync_copy(z1_h, cnta.at[pl.ds(slab, SLAB)])

    pltpu.sync_copy(one_h, onev)
    plsc.subcore_barrier()

    ebase = s * ET

    def batch_body(b, carry):
        eoff = ebase + b * B
        pltpu.sync_copy(src_h.at[pl.ds(eoff, B)], srcb)
        pltpu.sync_copy(dst_h.at[pl.ds(eoff, B)], dstb)
        for j in range(B // 16):
            sv = srcb[pl.ds(j * 16, 16)]
            dv = dstb[pl.ds(j * 16, 16)]
            m = (dv >= lo) & (dv < lo + NH)
            gidx[pl.ds(j * 16, 16)] = jnp.where(m, sv, 0)
            sidx[pl.ds(j * 16, 16)] = jnp.where(m, dv - lo, NH)
        pltpu.async_copy(x_h.at[gidx], rows, sem).wait()
        pltpu.sync_copy(rows, acc.at[sidx], add=True)
        pltpu.sync_copy(onev, cnta.at[sidx], add=True)
        return carry

    lax.fori_loop(0, NBATCH, batch_body, 0)
    plsc.subcore_barrier()

    g0 = lo + slab

    @pl.when(g0 + SLAB <= N)
    def _():
        pltpu.sync_copy(acc.at[pl.ds(slab, SLAB)], agg_h.at[pl.ds(g0, SLAB)])
        pltpu.sync_copy(cnta.at[pl.ds(slab, SLAB)], cnt_h.at[pl.ds(g0, SLAB)])

    @pl.when((g0 < N) & (g0 + SLAB > N))
    def _():
        def chunk_body(k, carry):
            gb = g0 + k * 16

            @pl.when(gb < N)
            def _():
                pltpu.sync_copy(acc.at[pl.ds(slab + k * 16, 16)],
                                agg_h.at[pl.ds(gb, 16)])
                pltpu.sync_copy(cnta.at[pl.ds(slab + k * 16, 16)],
                                cnt_h.at[pl.ds(gb, 16)])
            return carry

        lax.fori_loop(0, SLAB // 16, chunk_body, 0)


_sc_aggregate = pl.kernel(
    _sc_body,
    out_type=(
        jax.ShapeDtypeStruct((N, D), jnp.float32),
        jax.ShapeDtypeStruct((N,), jnp.float32),
    ),
    mesh=plsc.VectorSubcoreMesh(core_axis_name="c", subcore_axis_name="s"),
    scratch_types=[
        pltpu.VMEM((B,), jnp.int32),
        pltpu.VMEM((B,), jnp.int32),
        pltpu.VMEM((B,), jnp.int32),
        pltpu.VMEM((B,), jnp.int32),
        pltpu.VMEM((B, D), jnp.float32),
        pltpu.VMEM((B,), jnp.float32),
        pltpu.VMEM_SHARED((NH + 16, D), jnp.float32),
        pltpu.VMEM_SHARED((NH + 16,), jnp.float32),
        pltpu.SemaphoreType.DMA,
    ],
    compiler_params=pltpu.CompilerParams(use_tc_tiling_on_sc=False),
)


def _tc_body(agg_ref, cnt_ref, x_ref, wl_ref, bl_ref, wr_ref, out_ref, z_ref):
    inv = 1.0 / jnp.maximum(cnt_ref[...], 1.0)
    a = agg_ref[...] * inv
    o = (jnp.dot(a, wl_ref[...], preferred_element_type=jnp.float32)
         + bl_ref[...]
         + jnp.dot(x_ref[...], wr_ref[...], preferred_element_type=jnp.float32))
    out_ref[...] = o
    m = jnp.max(o, axis=1, keepdims=True)
    z_ref[...] = o - (m + jnp.log(jnp.sum(jnp.exp(o - m), axis=1,
                                          keepdims=True)))


BR = 2000


def _tc_combine(agg, cnt, x, w_l, b_l, w_r):
    grid = (N // BR,)
    return pl.pallas_call(
        _tc_body,
        grid=grid,
        in_specs=[
            pl.BlockSpec((BR, D), lambda i: (i, 0)),
            pl.BlockSpec((BR, 1), lambda i: (i, 0)),
            pl.BlockSpec((BR, D), lambda i: (i, 0)),
            pl.BlockSpec((D, D), lambda i: (0, 0)),
            pl.BlockSpec((1, D), lambda i: (0, 0)),
            pl.BlockSpec((D, D), lambda i: (0, 0)),
        ],
        out_specs=[
            pl.BlockSpec((BR, D), lambda i: (i, 0)),
            pl.BlockSpec((BR, D), lambda i: (i, 0)),
        ],
        out_shape=[
            jax.ShapeDtypeStruct((N, D), jnp.float32),
            jax.ShapeDtypeStruct((N, D), jnp.float32),
        ],
    )(agg, cnt, x, w_l, b_l, w_r)


@jax.jit
def kernel(x, adj_t, W_l, b_l, W_r):
    src = adj_t[0].astype(jnp.int32)
    dst = adj_t[1].astype(jnp.int32)
    pad = EPAD - E
    src_p = jnp.concatenate([src, jnp.zeros((pad,), jnp.int32)])
    dst_p = jnp.concatenate([dst, jnp.full((pad,), 2 * NH, jnp.int32)])
    z2 = jnp.zeros((SLAB, D), jnp.float32)
    z1 = jnp.zeros((SLAB,), jnp.float32)
    ones = jnp.ones((B,), jnp.float32)
    agg, cnt = _sc_aggregate(x, src_p, dst_p, z2, z1, ones)
    out, z = _tc_combine(agg, cnt.reshape(N, 1), x, W_l,
                         b_l.reshape(1, D), W_r)
    return (out, z)
